# Initial kernel scaffold; baseline (speedup 1.0000x reference)
import jax
import jax.numpy as jnp
from jax.experimental import pallas as pl


def kernel(x):
    raise NotImplementedError("write your pallas kernel here")


# speedup vs baseline: 12.0041x; 12.0041x over previous
import functools
import jax
import jax.numpy as jnp
from jax import lax
from jax.experimental import pallas as pl
from jax.experimental.pallas import tpu as pltpu
from jax.experimental.pallas import tpu_sc as plsc

A, B, N = 64, 32, 8192
ROWS = A * B
NW = 32
ROWS_PER_W = ROWS // NW
LANES = 16
NV = N // LANES
RANK = N - 1 - N // 2

MINI = -(2 ** 31)
MASK31 = 0x7FFFFFFF


def _splat(v, dtype=jnp.int32):
  return lax.broadcast(jnp.asarray(v, dtype), (LANES,))


@functools.partial(
    pl.kernel,
    out_type=jax.ShapeDtypeStruct((ROWS,), jnp.int32),
    mesh=plsc.VectorSubcoreMesh(core_axis_name="c", subcore_axis_name="s"),
    compiler_params=pltpu.CompilerParams(needs_layout_passes=False),
    scratch_types=[
        pltpu.VMEM((N,), jnp.int32),
        pltpu.VMEM((ROWS_PER_W,), jnp.int32),
    ],
)
def _select_kernel(x_hbm, out_hbm, key_v, res_v):
  cid = lax.axis_index("c")
  sid = lax.axis_index("s")
  wid = sid * 2 + cid
  base_row = wid * ROWS_PER_W
  lane = lax.broadcasted_iota(jnp.int32, (LANES,), 0)
  rank_s = _splat(RANK)

  def per_row(r, carry):
    pltpu.sync_copy(x_hbm.at[base_row + r], key_v)

    def xf(j, _):
      i = key_v[pl.ds(j * LANES, LANES)]
      key_v[pl.ds(j * LANES, LANES)] = jnp.where(i < 0, i ^ MASK31, i)
      return 0

    lax.fori_loop(0, NV, xf, 0, unroll=8)

    def per_bit(bi, pu):
      sh = _splat(31) - lax.broadcast(bi, (LANES,))
      bitv = lax.shift_left(_splat(1), sh)
      t_u = pu | bitv
      t_s = t_u ^ MINI

      def cnt(j, acc):
        kv = key_v[pl.ds(j * LANES, LANES)]
        return acc + plsc.all_reduce_population_count(kv < t_s)

      c = lax.fori_loop(0, NV, cnt, _splat(0), unroll=8)
      return jnp.where(c <= rank_s, t_u, pu)

    pu = lax.fori_loop(0, 32, per_bit, _splat(0))

    k_ans = pu ^ MINI
    i_ans = jnp.where(k_ans < 0, k_ans ^ MASK31, k_ans)
    plsc.store_scatter(res_v, [lax.broadcast(r, (LANES,))], i_ans,
                       mask=lane == 0)
    return carry

  lax.fori_loop(0, ROWS_PER_W, per_row, 0)
  pltpu.sync_copy(res_v, out_hbm.at[pl.ds(base_row, ROWS_PER_W)])


def kernel(x):
  bits = lax.bitcast_convert_type(x.reshape(ROWS, N), jnp.int32)
  out = _select_kernel(bits)
  return lax.bitcast_convert_type(out, jnp.float32).reshape(A, B)
